# Initial kernel scaffold; baseline (speedup 1.0000x reference)
import functools

import jax
import jax.numpy as jnp
from jax import lax
from jax.experimental import pallas as pl
from jax.experimental.pallas import tpu as pltpu
from jax.experimental.pallas import tpu_sc as plsc

VOCAB = 1000000
EMBED_DIM = 32
BATCH = 4096
SEQ_LEN = 200

N_IDX = BATCH * SEQ_LEN
NUM_CORES = 2
NUM_SUBCORES = 16
NW = NUM_CORES * NUM_SUBCORES
PER_W = N_IDX // NW
CHUNK = 128
NCH = PER_W // CHUNK

_MESH = plsc.VectorSubcoreMesh(
    core_axis_name="c", subcore_axis_name="s",
    num_cores=NUM_CORES, num_subcores=NUM_SUBCORES,
)


@functools.partial(
    pl.kernel,
    out_type=jax.ShapeDtypeStruct((N_IDX, EMBED_DIM), jnp.float32),
    mesh=_MESH,
    scratch_types=[
        pltpu.VMEM((NCH, CHUNK), jnp.int32),
        pltpu.VMEM((CHUNK, EMBED_DIM), jnp.float32),
        pltpu.SemaphoreType.DMA,
    ],
)
def _sc_gather(data_hbm, table_hbm, out_hbm, idx_v, rows_v, sem):
    wid = lax.axis_index("s") * NUM_CORES + lax.axis_index("c")
    base = wid * PER_W
    pltpu.sync_copy(data_hbm.at[wid], idx_v)

    @pl.loop(0, NCH)
    def _(j):
        pltpu.async_copy(table_hbm.at[idx_v.at[j]], rows_v, sem).wait()
        pltpu.sync_copy(rows_v, out_hbm.at[pl.ds(base + j * CHUNK, CHUNK)])


def kernel(data, table):
    flat = data.reshape(NW, NCH, CHUNK).astype(jnp.int32)
    out = _sc_gather(flat, table)
    return out.reshape(BATCH, SEQ_LEN, EMBED_DIM)


# speedup vs baseline: 1.3089x; 1.3089x over previous
import functools

import jax
import jax.numpy as jnp
from jax import lax
from jax.experimental import pallas as pl
from jax.experimental.pallas import tpu as pltpu
from jax.experimental.pallas import tpu_sc as plsc

VOCAB = 1000000
EMBED_DIM = 32
BATCH = 4096
SEQ_LEN = 200

N_IDX = BATCH * SEQ_LEN
NUM_CORES = 2
NUM_SUBCORES = 16
NW = NUM_CORES * NUM_SUBCORES
PER_W = N_IDX // NW
CHUNK = 128
NCH = PER_W // CHUNK

_MESH = plsc.VectorSubcoreMesh(
    core_axis_name="c", subcore_axis_name="s",
    num_cores=NUM_CORES, num_subcores=NUM_SUBCORES,
)


@functools.partial(
    pl.kernel,
    out_type=jax.ShapeDtypeStruct((N_IDX, EMBED_DIM), jnp.float32),
    mesh=_MESH,
    scratch_types=[
        pltpu.VMEM((NCH, CHUNK), jnp.int32),
        pltpu.VMEM((CHUNK, EMBED_DIM), jnp.float32),
        pltpu.SemaphoreType.DMA,
    ],
    compiler_params=pltpu.CompilerParams(use_tc_tiling_on_sc=False),
)
def _sc_gather(data_hbm, table_hbm, out_hbm, idx_v, rows_v, sem):
    wid = lax.axis_index("s") * NUM_CORES + lax.axis_index("c")
    base = wid * PER_W
    pltpu.sync_copy(data_hbm.at[wid], idx_v)

    @pl.loop(0, NCH)
    def _(j):
        pltpu.async_copy(table_hbm.at[idx_v.at[j]], rows_v, sem).wait()
        pltpu.sync_copy(rows_v, out_hbm.at[pl.ds(base + j * CHUNK, CHUNK)])


def kernel(data, table):
    flat = data.reshape(NW, NCH, CHUNK).astype(jnp.int32)
    out = _sc_gather(flat, table)
    return out.reshape(BATCH, SEQ_LEN, EMBED_DIM)


# speedup vs baseline: 1.5007x; 1.1466x over previous
import functools

import jax
import jax.numpy as jnp
from jax import lax
from jax.experimental import pallas as pl
from jax.experimental.pallas import tpu as pltpu
from jax.experimental.pallas import tpu_sc as plsc

VOCAB = 1000000
EMBED_DIM = 32
BATCH = 4096
SEQ_LEN = 200

N_IDX = BATCH * SEQ_LEN
NUM_CORES = 2
NUM_SUBCORES = 16
NW = NUM_CORES * NUM_SUBCORES
PER_W = N_IDX // NW
CHUNK = 128
NCH = PER_W // CHUNK
NBUF = 8
LOOKAHEAD = 4

_MESH = plsc.VectorSubcoreMesh(
    core_axis_name="c", subcore_axis_name="s",
    num_cores=NUM_CORES, num_subcores=NUM_SUBCORES,
)


@functools.partial(
    pl.kernel,
    out_type=jax.ShapeDtypeStruct((N_IDX, EMBED_DIM), jnp.float32),
    mesh=_MESH,
    scratch_types=[
        pltpu.VMEM((NCH, CHUNK), jnp.int32),
        pltpu.VMEM((NBUF, CHUNK, EMBED_DIM), jnp.float32),
        pltpu.SemaphoreType.DMA((NBUF,)),
        pltpu.SemaphoreType.DMA((NBUF,)),
    ],
    compiler_params=pltpu.CompilerParams(use_tc_tiling_on_sc=False),
)
def _sc_gather(data_hbm, table_hbm, out_hbm, idx_v, rows_v, gsem, wsem):
    wid = lax.axis_index("s") * NUM_CORES + lax.axis_index("c")
    base = wid * PER_W
    pltpu.sync_copy(data_hbm.at[wid], idx_v)

    for b in range(LOOKAHEAD):
        pltpu.async_copy(table_hbm.at[idx_v.at[b]], rows_v.at[b], gsem.at[b])

    @pl.loop(0, NCH)
    def _(j):
        b = lax.rem(j, NBUF)
        pltpu.make_async_copy(
            table_hbm.at[idx_v.at[j]], rows_v.at[b], gsem.at[b]).wait()
        pltpu.async_copy(
            rows_v.at[b], out_hbm.at[pl.ds(base + j * CHUNK, CHUNK)],
            wsem.at[b])

        jg = j + LOOKAHEAD
        bg = lax.rem(jg, NBUF)

        @pl.when(jg < NCH)
        def _():
            @pl.when(jg >= NBUF)
            def _():
                jw = jg - NBUF
                pltpu.make_async_copy(
                    rows_v.at[bg],
                    out_hbm.at[pl.ds(base + jw * CHUNK, CHUNK)],
                    wsem.at[bg]).wait()

            pltpu.async_copy(
                table_hbm.at[idx_v.at[jg]], rows_v.at[bg], gsem.at[bg])

    for t in range(NBUF):
        jw = NCH - NBUF + t
        b = jw % NBUF
        pltpu.make_async_copy(
            rows_v.at[b], out_hbm.at[pl.ds(base + jw * CHUNK, CHUNK)],
            wsem.at[b]).wait()


def kernel(data, table):
    flat = data.reshape(NW, NCH, CHUNK).astype(jnp.int32)
    out = _sc_gather(flat, table)
    return out.reshape(BATCH, SEQ_LEN, EMBED_DIM)


# speedup vs baseline: 1.5757x; 1.0500x over previous
import functools

import jax
import jax.numpy as jnp
from jax import lax
from jax.experimental import pallas as pl
from jax.experimental.pallas import tpu as pltpu
from jax.experimental.pallas import tpu_sc as plsc

VOCAB = 1000000
EMBED_DIM = 32
BATCH = 4096
SEQ_LEN = 200

N_IDX = BATCH * SEQ_LEN
NUM_CORES = 2
NUM_SUBCORES = 16
NW = NUM_CORES * NUM_SUBCORES
CHUNK = 128
NCH = SEQ_LEN
NBUF = 8
LOOKAHEAD = 4

_MESH = plsc.VectorSubcoreMesh(
    core_axis_name="c", subcore_axis_name="s",
    num_cores=NUM_CORES, num_subcores=NUM_SUBCORES,
)


@functools.partial(
    pl.kernel,
    out_type=jax.ShapeDtypeStruct((SEQ_LEN, BATCH, EMBED_DIM), jnp.float32),
    mesh=_MESH,
    scratch_types=[
        pltpu.VMEM((NCH, CHUNK), jnp.int32),
        pltpu.VMEM((NBUF, CHUNK, EMBED_DIM), jnp.float32),
        pltpu.SemaphoreType.DMA((NBUF,)),
        pltpu.SemaphoreType.DMA((NBUF,)),
    ],
    compiler_params=pltpu.CompilerParams(use_tc_tiling_on_sc=False),
)
def _sc_gather(data_hbm, table_hbm, out_hbm, idx_v, rows_v, gsem, wsem):
    wid = lax.axis_index("s") * NUM_CORES + lax.axis_index("c")
    b0 = wid * CHUNK
    pltpu.sync_copy(data_hbm.at[:, wid], idx_v)

    for b in range(LOOKAHEAD):
        pltpu.async_copy(table_hbm.at[idx_v.at[b]], rows_v.at[b], gsem.at[b])

    @pl.loop(0, NCH)
    def _(j):
        b = lax.rem(j, NBUF)
        pltpu.make_async_copy(
            table_hbm.at[idx_v.at[j]], rows_v.at[b], gsem.at[b]).wait()
        pltpu.async_copy(
            rows_v.at[b], out_hbm.at[j, pl.ds(b0, CHUNK)], wsem.at[b])

        jg = j + LOOKAHEAD
        bg = lax.rem(jg, NBUF)

        @pl.when(jg < NCH)
        def _():
            @pl.when(jg >= NBUF)
            def _():
                jw = jg - NBUF
                pltpu.make_async_copy(
                    rows_v.at[bg],
                    out_hbm.at[jw, pl.ds(b0, CHUNK)],
                    wsem.at[bg]).wait()

            pltpu.async_copy(
                table_hbm.at[idx_v.at[jg]], rows_v.at[bg], gsem.at[bg])

    for t in range(NBUF):
        jw = NCH - NBUF + t
        b = jw % NBUF
        pltpu.make_async_copy(
            rows_v.at[b], out_hbm.at[jw, pl.ds(b0, CHUNK)],
            wsem.at[b]).wait()


def kernel(data, table):
    data_sm = jnp.transpose(data).reshape(SEQ_LEN, NW, CHUNK).astype(jnp.int32)
    out_sm = _sc_gather(data_sm, table)
    return jnp.transpose(out_sm, (1, 0, 2))
